# baseline (speedup 1.0000x reference)
import functools

import jax
import jax.numpy as jnp
from jax import lax
from jax.experimental import pallas as pl
from jax.experimental.pallas import tpu as pltpu
from jax.experimental.pallas import tpu_sc as plsc

N, E, DF, DE, DO = 10000, 320000, 128, 16, 128
DC = DO + 16
NC, NS = 2, 16
NW = NC * NS
EPW = E // NW
BK = 80
NB = EPW // BK
RPT = N // NS


def _node_dense_body(x_ref, w1_ref, b1_ref, w2_ref, b2_ref, w4_ref, w5_ref,
                     b5_ref, A_ref, B_ref, C_ref, a_ref, bv_ref):
    x = x_ref[...]
    A = jnp.dot(x, w1_ref[...], preferred_element_type=jnp.float32) + b1_ref[...]
    B = jnp.dot(x, w2_ref[...], preferred_element_type=jnp.float32) + b2_ref[...]
    C = jnp.dot(x, w5_ref[...], preferred_element_type=jnp.float32) + b5_ref[...]
    A_ref[...] = A
    B_ref[...] = B
    C_ref[...] = C
    w4 = w4_ref[...]
    a_ref[...] = jnp.dot(A, w4, preferred_element_type=jnp.float32)
    bv_ref[...] = jnp.dot(B, w4, preferred_element_type=jnp.float32)


def _edge_dense_body(e_ref, w3_ref, b3_ref, w4_ref, b4_ref, EF0_ref, e0_ref):
    ew = e_ref[...]
    EF0 = jnp.dot(ew, w3_ref[...], preferred_element_type=jnp.float32) + b3_ref[...]
    EF0_ref[...] = EF0
    w34 = jnp.dot(w3_ref[...], w4_ref[...], preferred_element_type=jnp.float32)
    c34 = jnp.dot(b3_ref[...], w4_ref[...], preferred_element_type=jnp.float32)
    e0_ref[...] = (jnp.dot(ew, w34, preferred_element_type=jnp.float32)
                   + c34 + b4_ref[...])


def _combine_body(p0_ref, p1_ref, out_ref):
    acc = p0_ref[...] + p1_ref[...]
    num = acc[:, :DO]
    den = acc[:, DO:DO + 1]
    out_ref[...] = jnp.where(den > 0.0, num / den, 0.0)


def _sc_body(s3, r3, e3, a_hbm, b_hbm, A_hbm, B_hbm, Cw_hbm, EF0_hbm,
             ef_out, num_out,
             s_v, r_v, e_v, av, bv, ex_v, bufE, bufC, acc, sem1, sem2):
    cid = lax.axis_index("c")
    sid = lax.axis_index("s")
    wid = cid * NS + sid

    def _zrow(i, _):
        for q in range(DC // 16):
            bufC[i, pl.ds(q * 16, 16)] = jnp.zeros((16,), jnp.float32)
        return 0
    lax.fori_loop(0, BK, _zrow, 0)
    for t in range(RPT // BK):
        pltpu.sync_copy(bufC, acc.at[pl.ds(sid * RPT + t * BK, BK)])
    rem = RPT - (RPT // BK) * BK
    if rem:
        pltpu.sync_copy(bufC.at[pl.ds(0, rem)],
                        acc.at[pl.ds(sid * RPT + (RPT // BK) * BK, rem)])
    plsc.subcore_barrier()

    def _batch(b, _):
        g = wid * EPW + b * BK
        pltpu.async_copy(s3.at[wid, b], s_v, sem1)
        pltpu.async_copy(r3.at[wid, b], r_v, sem1)
        pltpu.async_copy(e3.at[wid, b], e_v, sem1)
        pltpu.async_copy(EF0_hbm.at[pl.ds(g, BK)], bufE, sem1)
        pltpu.make_async_copy(s3.at[wid, b], s_v, sem1).wait()
        pltpu.make_async_copy(r3.at[wid, b], r_v, sem1).wait()
        pltpu.make_async_copy(e3.at[wid, b], e_v, sem1).wait()
        pltpu.make_async_copy(EF0_hbm.at[pl.ds(g, BK)], bufE, sem1).wait()

        pltpu.async_copy(a_hbm.at[s_v], av, sem2)
        pltpu.async_copy(b_hbm.at[r_v], bv, sem2)
        pltpu.async_copy(Cw_hbm.at[s_v], bufC, sem2)
        pltpu.async_copy(A_hbm.at[s_v], bufE, sem2, add=True)
        pltpu.async_copy(B_hbm.at[r_v], bufE, sem2, add=True)
        pltpu.make_async_copy(a_hbm.at[s_v], av, sem2).wait()
        pltpu.make_async_copy(b_hbm.at[r_v], bv, sem2).wait()
        pltpu.make_async_copy(Cw_hbm.at[s_v], bufC, sem2).wait()
        pltpu.make_async_copy(A_hbm.at[s_v], bufE, sem2).wait()
        pltpu.make_async_copy(B_hbm.at[r_v], bufE, sem2).wait()

        for q in range(BK // 16):
            sl = pl.ds(q * 16, 16)
            att = e_v[sl] + av[sl] + bv[sl]
            att = jnp.where(att >= 0.0, att, 0.01 * att)
            ex_v[sl] = jnp.exp(att)

        def _scale(e, _):
            exb = plsc.load_gather(ex_v, [jnp.full((16,), e, jnp.int32)])
            for q in range(DC // 16):
                sl = pl.ds(q * 16, 16)
                bufC[e, sl] = bufC[e, sl] * exb
            return 0
        lax.fori_loop(0, BK, _scale, 0)

        pltpu.sync_copy(bufE, ef_out.at[pl.ds(g, BK)])
        pltpu.sync_copy(bufC, acc.at[r_v], add=True)
        return 0

    lax.fori_loop(0, NB, _batch, 0)
    plsc.subcore_barrier()

    pltpu.sync_copy(acc.at[pl.ds(sid * RPT, RPT)],
                    num_out.at[pl.ds(cid * N + sid * RPT, RPT)])


def _node_dense(nodes, W1, b1, W2, b2, W4, W5, b5):
    blk = 2000
    grid = (N // blk,)
    full = lambda shape: pl.BlockSpec(shape, lambda i: (0, 0))
    return pl.pallas_call(
        _node_dense_body,
        grid=grid,
        in_specs=[
            pl.BlockSpec((blk, DF), lambda i: (i, 0)),
            full((DF, DO)), full((1, DO)),
            full((DF, DO)), full((1, DO)),
            full((DO, 1)),
            full((DF, DO)), full((1, DO)),
        ],
        out_specs=[
            pl.BlockSpec((blk, DO), lambda i: (i, 0)),
            pl.BlockSpec((blk, DO), lambda i: (i, 0)),
            pl.BlockSpec((blk, DO), lambda i: (i, 0)),
            pl.BlockSpec((blk, 1), lambda i: (i, 0)),
            pl.BlockSpec((blk, 1), lambda i: (i, 0)),
        ],
        out_shape=[
            jax.ShapeDtypeStruct((N, DO), jnp.float32),
            jax.ShapeDtypeStruct((N, DO), jnp.float32),
            jax.ShapeDtypeStruct((N, DO), jnp.float32),
            jax.ShapeDtypeStruct((N, 1), jnp.float32),
            jax.ShapeDtypeStruct((N, 1), jnp.float32),
        ],
    )(nodes, W1, b1.reshape(1, DO), W2, b2.reshape(1, DO), W4, W5,
      b5.reshape(1, DO))


def _edge_dense(edges, W3, b3, W4, b4):
    blk = 3200
    grid = (E // blk,)
    full = lambda shape: pl.BlockSpec(shape, lambda i: (0, 0))
    return pl.pallas_call(
        _edge_dense_body,
        grid=grid,
        in_specs=[
            pl.BlockSpec((blk, DE), lambda i: (i, 0)),
            full((DE, DO)), full((1, DO)),
            full((DO, 1)), full((1, 1)),
        ],
        out_specs=[
            pl.BlockSpec((blk, DO), lambda i: (i, 0)),
            pl.BlockSpec((blk, 1), lambda i: (i, 0)),
        ],
        out_shape=[
            jax.ShapeDtypeStruct((E, DO), jnp.float32),
            jax.ShapeDtypeStruct((E, 1), jnp.float32),
        ],
    )(edges, W3, b3.reshape(1, DO), W4, b4.reshape(1, 1))


def _combine(num_flat):
    blk = 2000
    grid = (N // blk,)
    return pl.pallas_call(
        _combine_body,
        grid=grid,
        in_specs=[
            pl.BlockSpec((blk, DC), lambda i: (i, 0)),
            pl.BlockSpec((blk, DC), lambda i: (i + N // blk, 0)),
        ],
        out_specs=pl.BlockSpec((blk, DO), lambda i: (i, 0)),
        out_shape=jax.ShapeDtypeStruct((N, DO), jnp.float32),
    )(num_flat, num_flat)


@functools.cache
def _get_sc_edges():
    return pl.kernel(
        _sc_body,
        out_type=[
            jax.ShapeDtypeStruct((E, DO), jnp.float32),
            jax.ShapeDtypeStruct((NC * N, DC), jnp.float32),
        ],
        mesh=plsc.VectorSubcoreMesh(core_axis_name="c", subcore_axis_name="s"),
        scratch_types=[
            pltpu.VMEM((BK,), jnp.int32),
            pltpu.VMEM((BK,), jnp.int32),
            pltpu.VMEM((BK,), jnp.float32),
            pltpu.VMEM((BK,), jnp.float32),
            pltpu.VMEM((BK,), jnp.float32),
            pltpu.VMEM((BK,), jnp.float32),
            pltpu.VMEM((BK, DO), jnp.float32),
            pltpu.VMEM((BK, DC), jnp.float32),
            pltpu.VMEM_SHARED((N, DC), jnp.float32),
            pltpu.SemaphoreType.DMA,
            pltpu.SemaphoreType.DMA,
        ],
        compiler_params=pltpu.CompilerParams(use_tc_tiling_on_sc=False,
                                             needs_layout_passes=False),
    )


def kernel(nodes, edges, senders, receivers, W1, b1, W2, b2, W3, b3, W4, b4,
           W5, b5):
    A, B, C, a, bv = _node_dense(nodes, W1, b1, W2, b2, W4, W5, b5)
    EF0, e0 = _edge_dense(edges, W3, b3, W4, b4)

    Cw = jnp.concatenate(
        [C, jnp.ones((N, 1), jnp.float32), jnp.zeros((N, DC - DO - 1),
                                                     jnp.float32)], axis=1)

    s3 = senders.reshape(NW, NB, BK)
    r3 = receivers.reshape(NW, NB, BK)
    e3 = e0.reshape(NW, NB, BK)

    ef, num_flat = _get_sc_edges()(s3, r3, e3, a.reshape(N), bv.reshape(N),
                                   A, B, Cw, EF0)
    new_nodes = _combine(num_flat)
    return new_nodes, ef


# speedup vs baseline: 1.0959x; 1.0959x over previous
import functools

import jax
import jax.numpy as jnp
from jax import lax
from jax.experimental import pallas as pl
from jax.experimental.pallas import tpu as pltpu
from jax.experimental.pallas import tpu_sc as plsc

N, E, DF, DE, DO = 10000, 320000, 128, 16, 128
DC = DO + 16
NC, NS = 2, 16
NW = NC * NS
EPW = E // NW
BK = 80
NB = EPW // BK
RPT = N // NS


def _node_dense_body(x_ref, w1_ref, b1_ref, w2_ref, b2_ref, w4_ref, w5_ref,
                     b5_ref, A_ref, B_ref, C_ref, a_ref, bv_ref):
    x = x_ref[...]
    A = jnp.dot(x, w1_ref[...], preferred_element_type=jnp.float32) + b1_ref[...]
    B = jnp.dot(x, w2_ref[...], preferred_element_type=jnp.float32) + b2_ref[...]
    C = jnp.dot(x, w5_ref[...], preferred_element_type=jnp.float32) + b5_ref[...]
    A_ref[...] = A
    B_ref[...] = B
    C_ref[...] = C
    w4 = w4_ref[...]
    a_ref[...] = jnp.dot(A, w4, preferred_element_type=jnp.float32)
    bv_ref[...] = jnp.dot(B, w4, preferred_element_type=jnp.float32)


def _edge_dense_body(e_ref, w3_ref, b3_ref, w4_ref, b4_ref, EF0_ref, e0_ref):
    ew = e_ref[...]
    EF0 = jnp.dot(ew, w3_ref[...], preferred_element_type=jnp.float32) + b3_ref[...]
    EF0_ref[...] = EF0
    w34 = jnp.dot(w3_ref[...], w4_ref[...], preferred_element_type=jnp.float32)
    c34 = jnp.dot(b3_ref[...], w4_ref[...], preferred_element_type=jnp.float32)
    e0_ref[...] = (jnp.dot(ew, w34, preferred_element_type=jnp.float32)
                   + c34 + b4_ref[...])


def _combine_body(p0_ref, p1_ref, out_ref):
    acc = p0_ref[...] + p1_ref[...]
    num = acc[:, :DO]
    den = acc[:, DO:DO + 1]
    out_ref[...] = jnp.where(den > 0.0, num / den, 0.0)


def _sc_body(s3, r3, e3, a_hbm, b_hbm, A_hbm, B_hbm, Cw_hbm, EF0_hbm,
             ef_out, num_out,
             s_v, r_v, e_v, av, bv, ex_v, bufE, bufC, acc, sem1, sem2):
    cid = lax.axis_index("c")
    sid = lax.axis_index("s")
    wid = cid * NS + sid

    def _zrow(i, _):
        for q in range(DC // 16):
            bufC[i, pl.ds(q * 16, 16)] = jnp.zeros((16,), jnp.float32)
        return 0
    lax.fori_loop(0, BK, _zrow, 0)
    for t in range(RPT // BK):
        pltpu.sync_copy(bufC, acc.at[pl.ds(sid * RPT + t * BK, BK)])
    rem = RPT - (RPT // BK) * BK
    if rem:
        pltpu.sync_copy(bufC.at[pl.ds(0, rem)],
                        acc.at[pl.ds(sid * RPT + (RPT // BK) * BK, rem)])
    plsc.subcore_barrier()

    def issue1(n, slot):
        g = wid * EPW + n * BK
        pltpu.async_copy(s3.at[wid, n], s_v.at[slot], sem1.at[slot])
        pltpu.async_copy(r3.at[wid, n], r_v.at[slot], sem1.at[slot])
        pltpu.async_copy(e3.at[wid, n], e_v.at[slot], sem1.at[slot])
        pltpu.async_copy(EF0_hbm.at[pl.ds(g, BK)], bufE.at[slot],
                         sem1.at[slot])

    def wait1(n, slot):
        g = wid * EPW + n * BK
        pltpu.make_async_copy(s3.at[wid, n], s_v.at[slot],
                              sem1.at[slot]).wait()
        pltpu.make_async_copy(r3.at[wid, n], r_v.at[slot],
                              sem1.at[slot]).wait()
        pltpu.make_async_copy(e3.at[wid, n], e_v.at[slot],
                              sem1.at[slot]).wait()
        pltpu.make_async_copy(EF0_hbm.at[pl.ds(g, BK)], bufE.at[slot],
                              sem1.at[slot]).wait()

    issue1(0, 0)

    def _batch(b, _):
        S = b % 2
        wait1(b, S)
        issue1(jnp.minimum(b + 1, NB - 1), (b + 1) % 2)

        sv = s_v.at[S]
        rv = r_v.at[S]
        be = bufE.at[S]
        pltpu.async_copy(a_hbm.at[sv], av, sem2)
        pltpu.async_copy(b_hbm.at[rv], bv, sem2)
        pltpu.async_copy(Cw_hbm.at[sv], bufC, sem2)
        pltpu.async_copy(A_hbm.at[sv], be, sem2, add=True)
        pltpu.async_copy(B_hbm.at[rv], be, sem2, add=True)
        pltpu.make_async_copy(a_hbm.at[sv], av, sem2).wait()
        pltpu.make_async_copy(b_hbm.at[rv], bv, sem2).wait()
        pltpu.make_async_copy(Cw_hbm.at[sv], bufC, sem2).wait()
        pltpu.make_async_copy(A_hbm.at[sv], be, sem2).wait()
        pltpu.make_async_copy(B_hbm.at[rv], be, sem2).wait()

        ev = e_v.at[S]
        for q in range(BK // 16):
            sl = pl.ds(q * 16, 16)
            att = ev[sl] + av[sl] + bv[sl]
            att = jnp.where(att >= 0.0, att, 0.01 * att)
            ex_v[sl] = jnp.exp(att)

        def _scale(e, _):
            exb = plsc.load_gather(ex_v, [jnp.full((16,), e, jnp.int32)])
            for q in range(DC // 16):
                sl = pl.ds(q * 16, 16)
                bufC[e, sl] = bufC[e, sl] * exb
            return 0
        lax.fori_loop(0, BK, _scale, 0)

        g = wid * EPW + b * BK
        pltpu.sync_copy(be, ef_out.at[pl.ds(g, BK)])
        pltpu.sync_copy(bufC, acc.at[rv], add=True)
        return 0

    lax.fori_loop(0, NB, _batch, 0)
    wait1(NB - 1, NB % 2)
    plsc.subcore_barrier()

    pltpu.sync_copy(acc.at[pl.ds(sid * RPT, RPT)],
                    num_out.at[pl.ds(cid * N + sid * RPT, RPT)])


def _node_dense(nodes, W1, b1, W2, b2, W4, W5, b5):
    blk = 2000
    grid = (N // blk,)
    full = lambda shape: pl.BlockSpec(shape, lambda i: (0, 0))
    return pl.pallas_call(
        _node_dense_body,
        grid=grid,
        in_specs=[
            pl.BlockSpec((blk, DF), lambda i: (i, 0)),
            full((DF, DO)), full((1, DO)),
            full((DF, DO)), full((1, DO)),
            full((DO, 1)),
            full((DF, DO)), full((1, DO)),
        ],
        out_specs=[
            pl.BlockSpec((blk, DO), lambda i: (i, 0)),
            pl.BlockSpec((blk, DO), lambda i: (i, 0)),
            pl.BlockSpec((blk, DO), lambda i: (i, 0)),
            pl.BlockSpec((blk, 1), lambda i: (i, 0)),
            pl.BlockSpec((blk, 1), lambda i: (i, 0)),
        ],
        out_shape=[
            jax.ShapeDtypeStruct((N, DO), jnp.float32),
            jax.ShapeDtypeStruct((N, DO), jnp.float32),
            jax.ShapeDtypeStruct((N, DO), jnp.float32),
            jax.ShapeDtypeStruct((N, 1), jnp.float32),
            jax.ShapeDtypeStruct((N, 1), jnp.float32),
        ],
    )(nodes, W1, b1.reshape(1, DO), W2, b2.reshape(1, DO), W4, W5,
      b5.reshape(1, DO))


def _edge_dense(edges, W3, b3, W4, b4):
    blk = 3200
    grid = (E // blk,)
    full = lambda shape: pl.BlockSpec(shape, lambda i: (0, 0))
    return pl.pallas_call(
        _edge_dense_body,
        grid=grid,
        in_specs=[
            pl.BlockSpec((blk, DE), lambda i: (i, 0)),
            full((DE, DO)), full((1, DO)),
            full((DO, 1)), full((1, 1)),
        ],
        out_specs=[
            pl.BlockSpec((blk, DO), lambda i: (i, 0)),
            pl.BlockSpec((blk, 1), lambda i: (i, 0)),
        ],
        out_shape=[
            jax.ShapeDtypeStruct((E, DO), jnp.float32),
            jax.ShapeDtypeStruct((E, 1), jnp.float32),
        ],
    )(edges, W3, b3.reshape(1, DO), W4, b4.reshape(1, 1))


def _combine(num_flat):
    blk = 2000
    grid = (N // blk,)
    return pl.pallas_call(
        _combine_body,
        grid=grid,
        in_specs=[
            pl.BlockSpec((blk, DC), lambda i: (i, 0)),
            pl.BlockSpec((blk, DC), lambda i: (i + N // blk, 0)),
        ],
        out_specs=pl.BlockSpec((blk, DO), lambda i: (i, 0)),
        out_shape=jax.ShapeDtypeStruct((N, DO), jnp.float32),
    )(num_flat, num_flat)


@functools.cache
def _get_sc_edges():
    return pl.kernel(
        _sc_body,
        out_type=[
            jax.ShapeDtypeStruct((E, DO), jnp.float32),
            jax.ShapeDtypeStruct((NC * N, DC), jnp.float32),
        ],
        mesh=plsc.VectorSubcoreMesh(core_axis_name="c", subcore_axis_name="s"),
        scratch_types=[
            pltpu.VMEM((2, BK), jnp.int32),
            pltpu.VMEM((2, BK), jnp.int32),
            pltpu.VMEM((2, BK), jnp.float32),
            pltpu.VMEM((BK,), jnp.float32),
            pltpu.VMEM((BK,), jnp.float32),
            pltpu.VMEM((BK,), jnp.float32),
            pltpu.VMEM((2, BK, DO), jnp.float32),
            pltpu.VMEM((BK, DC), jnp.float32),
            pltpu.VMEM_SHARED((N, DC), jnp.float32),
            pltpu.SemaphoreType.DMA((2,)),
            pltpu.SemaphoreType.DMA,
        ],
        compiler_params=pltpu.CompilerParams(use_tc_tiling_on_sc=False,
                                             needs_layout_passes=False),
    )


def kernel(nodes, edges, senders, receivers, W1, b1, W2, b2, W3, b3, W4, b4,
           W5, b5):
    A, B, C, a, bv = _node_dense(nodes, W1, b1, W2, b2, W4, W5, b5)
    EF0, e0 = _edge_dense(edges, W3, b3, W4, b4)

    Cw = jnp.concatenate(
        [C, jnp.ones((N, 1), jnp.float32), jnp.zeros((N, DC - DO - 1),
                                                     jnp.float32)], axis=1)

    s3 = senders.reshape(NW, NB, BK)
    r3 = receivers.reshape(NW, NB, BK)
    e3 = e0.reshape(NW, NB, BK)

    ef, num_flat = _get_sc_edges()(s3, r3, e3, a.reshape(N), bv.reshape(N),
                                   A, B, Cw, EF0)
    new_nodes = _combine(num_flat)
    return new_nodes, ef


# speedup vs baseline: 1.1423x; 1.0424x over previous
import functools

import jax
import jax.numpy as jnp
from jax import lax
from jax.experimental import pallas as pl
from jax.experimental.pallas import tpu as pltpu
from jax.experimental.pallas import tpu_sc as plsc

N, E, DF, DE, DO = 10000, 320000, 128, 16, 128
DC = DO + 16
NC, NS = 2, 16
NW = NC * NS
EPW = E // NW
BK = 80
NB = EPW // BK
RPT = N // NS


def _node_dense_body(x_ref, w1_ref, b1_ref, w2_ref, b2_ref, w4_ref, w5_ref,
                     b5_ref, A_ref, B_ref, C_ref, a_ref, bv_ref):
    x = x_ref[...]
    A = jnp.dot(x, w1_ref[...], preferred_element_type=jnp.float32) + b1_ref[...]
    B = jnp.dot(x, w2_ref[...], preferred_element_type=jnp.float32) + b2_ref[...]
    C = jnp.dot(x, w5_ref[...], preferred_element_type=jnp.float32) + b5_ref[...]
    A_ref[...] = A
    B_ref[...] = B
    C_ref[...] = C
    w4 = w4_ref[...]
    a_ref[...] = jnp.dot(A, w4, preferred_element_type=jnp.float32)
    bv_ref[...] = jnp.dot(B, w4, preferred_element_type=jnp.float32)


def _edge_dense_body(e_ref, w3_ref, b3_ref, w4_ref, b4_ref, EF0_ref, e0_ref):
    ew = e_ref[...]
    EF0 = jnp.dot(ew, w3_ref[...], preferred_element_type=jnp.float32) + b3_ref[...]
    EF0_ref[...] = EF0
    w34 = jnp.dot(w3_ref[...], w4_ref[...], preferred_element_type=jnp.float32)
    c34 = jnp.dot(b3_ref[...], w4_ref[...], preferred_element_type=jnp.float32)
    e0_ref[...] = (jnp.dot(ew, w34, preferred_element_type=jnp.float32)
                   + c34 + b4_ref[...])


def _combine_body(p0_ref, p1_ref, out_ref):
    acc = p0_ref[...] + p1_ref[...]
    num = acc[:, :DO]
    den = acc[:, DO:DO + 1]
    out_ref[...] = jnp.where(den > 0.0, num / den, 0.0)


def _sc_body(s3, r3, e3, a_hbm, b_hbm, A_hbm, B_hbm, Cw_hbm, EF0_hbm,
             ef_out, num_out,
             s_v, r_v, e_v, av, bv, ex_v, bufE, bufC, acc, sem1, sem2,
             semC, semE):
    cid = lax.axis_index("c")
    sid = lax.axis_index("s")
    wid = cid * NS + sid

    def _zrow(i, _):
        for q in range(DC // 16):
            bufC[i, pl.ds(q * 16, 16)] = jnp.zeros((16,), jnp.float32)
        return 0
    lax.fori_loop(0, BK, _zrow, 0)
    for t in range(RPT // BK):
        pltpu.sync_copy(bufC, acc.at[pl.ds(sid * RPT + t * BK, BK)])
    rem = RPT - (RPT // BK) * BK
    if rem:
        pltpu.sync_copy(bufC.at[pl.ds(0, rem)],
                        acc.at[pl.ds(sid * RPT + (RPT // BK) * BK, rem)])
    plsc.subcore_barrier()

    def issue1(n, slot):
        g = wid * EPW + n * BK
        pltpu.async_copy(s3.at[wid, n], s_v.at[slot], sem1.at[slot])
        pltpu.async_copy(r3.at[wid, n], r_v.at[slot], sem1.at[slot])
        pltpu.async_copy(e3.at[wid, n], e_v.at[slot], sem1.at[slot])
        pltpu.async_copy(EF0_hbm.at[pl.ds(g, BK)], bufE.at[slot],
                         sem1.at[slot])

    def wait1(n, slot):
        g = wid * EPW + n * BK
        pltpu.make_async_copy(s3.at[wid, n], s_v.at[slot],
                              sem1.at[slot]).wait()
        pltpu.make_async_copy(r3.at[wid, n], r_v.at[slot],
                              sem1.at[slot]).wait()
        pltpu.make_async_copy(e3.at[wid, n], e_v.at[slot],
                              sem1.at[slot]).wait()
        pltpu.make_async_copy(EF0_hbm.at[pl.ds(g, BK)], bufE.at[slot],
                              sem1.at[slot]).wait()

    issue1(0, 0)

    def _batch(b, _):
        S = b % 2
        wait1(b, S)
        issue1(jnp.minimum(b + 1, NB - 1), (b + 1) % 2)

        sv = s_v.at[S]
        rv = r_v.at[S]
        be = bufE.at[S]
        pltpu.async_copy(a_hbm.at[sv], av, sem2)
        pltpu.async_copy(b_hbm.at[rv], bv, sem2)
        pltpu.async_copy(Cw_hbm.at[sv], bufC, semC)
        pltpu.async_copy(A_hbm.at[sv], be, semE, add=True)
        pltpu.async_copy(B_hbm.at[rv], be, semE, add=True)
        pltpu.make_async_copy(a_hbm.at[sv], av, sem2).wait()
        pltpu.make_async_copy(b_hbm.at[rv], bv, sem2).wait()

        ev = e_v.at[S]
        for q in range(BK // 16):
            sl = pl.ds(q * 16, 16)
            att = ev[sl] + av[sl] + bv[sl]
            att = jnp.where(att >= 0.0, att, 0.01 * att)
            ex_v[sl] = jnp.exp(att)

        pltpu.make_async_copy(Cw_hbm.at[sv], bufC, semC).wait()

        def _scale(e, _):
            exb = plsc.load_gather(ex_v, [jnp.full((16,), e, jnp.int32)])
            for q in range(DC // 16):
                sl = pl.ds(q * 16, 16)
                bufC[e, sl] = bufC[e, sl] * exb
            return 0
        lax.fori_loop(0, BK, _scale, 0)

        pltpu.make_async_copy(A_hbm.at[sv], be, semE).wait()
        pltpu.make_async_copy(B_hbm.at[rv], be, semE).wait()
        g = wid * EPW + b * BK
        pltpu.sync_copy(be, ef_out.at[pl.ds(g, BK)])
        pltpu.sync_copy(bufC, acc.at[rv], add=True)
        return 0

    lax.fori_loop(0, NB, _batch, 0)
    wait1(NB - 1, NB % 2)
    plsc.subcore_barrier()

    pltpu.sync_copy(acc.at[pl.ds(sid * RPT, RPT)],
                    num_out.at[pl.ds(cid * N + sid * RPT, RPT)])


def _node_dense(nodes, W1, b1, W2, b2, W4, W5, b5):
    blk = 2000
    grid = (N // blk,)
    full = lambda shape: pl.BlockSpec(shape, lambda i: (0, 0))
    return pl.pallas_call(
        _node_dense_body,
        grid=grid,
        in_specs=[
            pl.BlockSpec((blk, DF), lambda i: (i, 0)),
            full((DF, DO)), full((1, DO)),
            full((DF, DO)), full((1, DO)),
            full((DO, 1)),
            full((DF, DO)), full((1, DO)),
        ],
        out_specs=[
            pl.BlockSpec((blk, DO), lambda i: (i, 0)),
            pl.BlockSpec((blk, DO), lambda i: (i, 0)),
            pl.BlockSpec((blk, DO), lambda i: (i, 0)),
            pl.BlockSpec((blk, 1), lambda i: (i, 0)),
            pl.BlockSpec((blk, 1), lambda i: (i, 0)),
        ],
        out_shape=[
            jax.ShapeDtypeStruct((N, DO), jnp.float32),
            jax.ShapeDtypeStruct((N, DO), jnp.float32),
            jax.ShapeDtypeStruct((N, DO), jnp.float32),
            jax.ShapeDtypeStruct((N, 1), jnp.float32),
            jax.ShapeDtypeStruct((N, 1), jnp.float32),
        ],
    )(nodes, W1, b1.reshape(1, DO), W2, b2.reshape(1, DO), W4, W5,
      b5.reshape(1, DO))


def _edge_dense(edges, W3, b3, W4, b4):
    blk = 3200
    grid = (E // blk,)
    full = lambda shape: pl.BlockSpec(shape, lambda i: (0, 0))
    return pl.pallas_call(
        _edge_dense_body,
        grid=grid,
        in_specs=[
            pl.BlockSpec((blk, DE), lambda i: (i, 0)),
            full((DE, DO)), full((1, DO)),
            full((DO, 1)), full((1, 1)),
        ],
        out_specs=[
            pl.BlockSpec((blk, DO), lambda i: (i, 0)),
            pl.BlockSpec((blk, 1), lambda i: (i, 0)),
        ],
        out_shape=[
            jax.ShapeDtypeStruct((E, DO), jnp.float32),
            jax.ShapeDtypeStruct((E, 1), jnp.float32),
        ],
    )(edges, W3, b3.reshape(1, DO), W4, b4.reshape(1, 1))


def _combine(num_flat):
    blk = 2000
    grid = (N // blk,)
    return pl.pallas_call(
        _combine_body,
        grid=grid,
        in_specs=[
            pl.BlockSpec((blk, DC), lambda i: (i, 0)),
            pl.BlockSpec((blk, DC), lambda i: (i + N // blk, 0)),
        ],
        out_specs=pl.BlockSpec((blk, DO), lambda i: (i, 0)),
        out_shape=jax.ShapeDtypeStruct((N, DO), jnp.float32),
    )(num_flat, num_flat)


@functools.cache
def _get_sc_edges():
    return pl.kernel(
        _sc_body,
        out_type=[
            jax.ShapeDtypeStruct((E, DO), jnp.float32),
            jax.ShapeDtypeStruct((NC * N, DC), jnp.float32),
        ],
        mesh=plsc.VectorSubcoreMesh(core_axis_name="c", subcore_axis_name="s"),
        scratch_types=[
            pltpu.VMEM((2, BK), jnp.int32),
            pltpu.VMEM((2, BK), jnp.int32),
            pltpu.VMEM((2, BK), jnp.float32),
            pltpu.VMEM((BK,), jnp.float32),
            pltpu.VMEM((BK,), jnp.float32),
            pltpu.VMEM((BK,), jnp.float32),
            pltpu.VMEM((2, BK, DO), jnp.float32),
            pltpu.VMEM((BK, DC), jnp.float32),
            pltpu.VMEM_SHARED((N, DC), jnp.float32),
            pltpu.SemaphoreType.DMA((2,)),
            pltpu.SemaphoreType.DMA,
            pltpu.SemaphoreType.DMA,
            pltpu.SemaphoreType.DMA,
        ],
        compiler_params=pltpu.CompilerParams(use_tc_tiling_on_sc=False,
                                             needs_layout_passes=False),
    )


def kernel(nodes, edges, senders, receivers, W1, b1, W2, b2, W3, b3, W4, b4,
           W5, b5):
    A, B, C, a, bv = _node_dense(nodes, W1, b1, W2, b2, W4, W5, b5)
    EF0, e0 = _edge_dense(edges, W3, b3, W4, b4)

    Cw = jnp.concatenate(
        [C, jnp.ones((N, 1), jnp.float32), jnp.zeros((N, DC - DO - 1),
                                                     jnp.float32)], axis=1)

    s3 = senders.reshape(NW, NB, BK)
    r3 = receivers.reshape(NW, NB, BK)
    e3 = e0.reshape(NW, NB, BK)

    ef, num_flat = _get_sc_edges()(s3, r3, e3, a.reshape(N), bv.reshape(N),
                                   A, B, Cw, EF0)
    new_nodes = _combine(num_flat)
    return new_nodes, ef


# speedup vs baseline: 1.1947x; 1.0459x over previous
import functools

import jax
import jax.numpy as jnp
from jax import lax
from jax.experimental import pallas as pl
from jax.experimental.pallas import tpu as pltpu
from jax.experimental.pallas import tpu_sc as plsc

N, E, DF, DE, DO = 10000, 320000, 128, 16, 128
DC = DO + 16
NC, NS = 2, 16
NW = NC * NS
EPW = E // NW
BK = 80
NB = EPW // BK
RPT = N // NS


def _node_dense_body(x_ref, w1_ref, b1_ref, w2_ref, b2_ref, w4_ref, w5_ref,
                     b5_ref, A_ref, B_ref, C_ref, a_ref, bv_ref):
    x = x_ref[...]
    A = jnp.dot(x, w1_ref[...], preferred_element_type=jnp.float32) + b1_ref[...]
    B = jnp.dot(x, w2_ref[...], preferred_element_type=jnp.float32) + b2_ref[...]
    C = jnp.dot(x, w5_ref[...], preferred_element_type=jnp.float32) + b5_ref[...]
    A_ref[...] = A
    B_ref[...] = B
    C_ref[...] = C
    w4 = w4_ref[...]
    a_ref[...] = jnp.dot(A, w4, preferred_element_type=jnp.float32)
    bv_ref[...] = jnp.dot(B, w4, preferred_element_type=jnp.float32)


def _edge_dense_body(e_ref, w3_ref, b3_ref, w4_ref, b4_ref, EF0_ref, e0_ref):
    ew = e_ref[...]
    EF0 = jnp.dot(ew, w3_ref[...], preferred_element_type=jnp.float32) + b3_ref[...]
    EF0_ref[...] = EF0
    w34 = jnp.dot(w3_ref[...], w4_ref[...], preferred_element_type=jnp.float32)
    c34 = jnp.dot(b3_ref[...], w4_ref[...], preferred_element_type=jnp.float32)
    e0_ref[...] = (jnp.dot(ew, w34, preferred_element_type=jnp.float32)
                   + c34 + b4_ref[...])


def _combine_body(p0_ref, p1_ref, out_ref):
    acc = p0_ref[...] + p1_ref[...]
    num = acc[:, :DO]
    den = acc[:, DO:DO + 1]
    out_ref[...] = jnp.where(den > 0.0, num / den, 0.0)


def _sc_body(s3, r3, e3, a_hbm, b_hbm, A_hbm, B_hbm, Cw_hbm, EF0_hbm,
             ef_out, num_out,
             s_v, r_v, e_v, av, bv, ex_v, bufE, bufC, acc, sem1, sem2,
             semC, semE):
    cid = lax.axis_index("c")
    sid = lax.axis_index("s")
    wid = cid * NS + sid

    def _zrow(i, _):
        for q in range(DC // 16):
            bufC[i, pl.ds(q * 16, 16)] = jnp.zeros((16,), jnp.float32)
        return 0
    lax.fori_loop(0, BK, _zrow, 0)
    for t in range(RPT // BK):
        pltpu.sync_copy(bufC, acc.at[pl.ds(sid * RPT + t * BK, BK)])
    rem = RPT - (RPT // BK) * BK
    if rem:
        pltpu.sync_copy(bufC.at[pl.ds(0, rem)],
                        acc.at[pl.ds(sid * RPT + (RPT // BK) * BK, rem)])
    plsc.subcore_barrier()

    def issue1(n, slot):
        g = wid * EPW + n * BK
        pltpu.async_copy(s3.at[wid, n], s_v.at[slot], sem1.at[slot])
        pltpu.async_copy(r3.at[wid, n], r_v.at[slot], sem1.at[slot])
        pltpu.async_copy(e3.at[wid, n], e_v.at[slot], sem1.at[slot])
        pltpu.async_copy(EF0_hbm.at[pl.ds(g, BK)], bufE.at[slot],
                         sem1.at[slot])

    def wait1(n, slot):
        g = wid * EPW + n * BK
        pltpu.make_async_copy(s3.at[wid, n], s_v.at[slot],
                              sem1.at[slot]).wait()
        pltpu.make_async_copy(r3.at[wid, n], r_v.at[slot],
                              sem1.at[slot]).wait()
        pltpu.make_async_copy(e3.at[wid, n], e_v.at[slot],
                              sem1.at[slot]).wait()
        pltpu.make_async_copy(EF0_hbm.at[pl.ds(g, BK)], bufE.at[slot],
                              sem1.at[slot]).wait()

    issue1(0, 0)

    def _batch(b, _):
        S = b % 2
        wait1(b, S)
        issue1(jnp.minimum(b + 1, NB - 1), (b + 1) % 2)

        sv = s_v.at[S]
        rv = r_v.at[S]
        be = bufE.at[S]
        pltpu.async_copy(a_hbm.at[sv], av, sem2)
        pltpu.async_copy(b_hbm.at[rv], bv, sem2)
        pltpu.async_copy(Cw_hbm.at[sv], bufC, semC)
        pltpu.async_copy(A_hbm.at[sv], be, semE, add=True)
        pltpu.async_copy(B_hbm.at[rv], be, semE, add=True)
        pltpu.make_async_copy(a_hbm.at[sv], av, sem2).wait()
        pltpu.make_async_copy(b_hbm.at[rv], bv, sem2).wait()

        ev = e_v.at[S]
        for q in range(BK // 16):
            sl = pl.ds(q * 16, 16)
            att = ev[sl] + av[sl] + bv[sl]
            att = jnp.where(att >= 0.0, att, 0.01 * att)
            ex_v[sl] = jnp.exp(att)

        pltpu.make_async_copy(Cw_hbm.at[sv], bufC, semC).wait()

        def _scale(e, _):
            exb = plsc.load_gather(ex_v, [jnp.full((16,), e, jnp.int32)])
            for q in range(DC // 16):
                sl = pl.ds(q * 16, 16)
                bufC[e, sl] = bufC[e, sl] * exb
            return 0
        lax.fori_loop(0, BK, _scale, 0)

        pltpu.make_async_copy(A_hbm.at[sv], be, semE).wait()
        pltpu.make_async_copy(B_hbm.at[rv], be, semE).wait()
        g = wid * EPW + b * BK
        pltpu.async_copy(be, ef_out.at[pl.ds(g, BK)], semE)
        pltpu.sync_copy(bufC, acc.at[rv], add=True)
        pltpu.make_async_copy(be, ef_out.at[pl.ds(g, BK)], semE).wait()
        return 0

    lax.fori_loop(0, NB, _batch, 0)
    wait1(NB - 1, NB % 2)
    plsc.subcore_barrier()

    pltpu.sync_copy(acc.at[pl.ds(sid * RPT, RPT)],
                    num_out.at[pl.ds(cid * N + sid * RPT, RPT)])


def _node_dense(nodes, W1, b1, W2, b2, W4, W5, b5):
    blk = 2000
    grid = (N // blk,)
    full = lambda shape: pl.BlockSpec(shape, lambda i: (0, 0))
    return pl.pallas_call(
        _node_dense_body,
        grid=grid,
        in_specs=[
            pl.BlockSpec((blk, DF), lambda i: (i, 0)),
            full((DF, DO)), full((1, DO)),
            full((DF, DO)), full((1, DO)),
            full((DO, 1)),
            full((DF, DO)), full((1, DO)),
        ],
        out_specs=[
            pl.BlockSpec((blk, DO), lambda i: (i, 0)),
            pl.BlockSpec((blk, DO), lambda i: (i, 0)),
            pl.BlockSpec((blk, DO), lambda i: (i, 0)),
            pl.BlockSpec((blk, 1), lambda i: (i, 0)),
            pl.BlockSpec((blk, 1), lambda i: (i, 0)),
        ],
        out_shape=[
            jax.ShapeDtypeStruct((N, DO), jnp.float32),
            jax.ShapeDtypeStruct((N, DO), jnp.float32),
            jax.ShapeDtypeStruct((N, DO), jnp.float32),
            jax.ShapeDtypeStruct((N, 1), jnp.float32),
            jax.ShapeDtypeStruct((N, 1), jnp.float32),
        ],
    )(nodes, W1, b1.reshape(1, DO), W2, b2.reshape(1, DO), W4, W5,
      b5.reshape(1, DO))


def _edge_dense(edges, W3, b3, W4, b4):
    blk = 3200
    grid = (E // blk,)
    full = lambda shape: pl.BlockSpec(shape, lambda i: (0, 0))
    return pl.pallas_call(
        _edge_dense_body,
        grid=grid,
        in_specs=[
            pl.BlockSpec((blk, DE), lambda i: (i, 0)),
            full((DE, DO)), full((1, DO)),
            full((DO, 1)), full((1, 1)),
        ],
        out_specs=[
            pl.BlockSpec((blk, DO), lambda i: (i, 0)),
            pl.BlockSpec((blk, 1), lambda i: (i, 0)),
        ],
        out_shape=[
            jax.ShapeDtypeStruct((E, DO), jnp.float32),
            jax.ShapeDtypeStruct((E, 1), jnp.float32),
        ],
    )(edges, W3, b3.reshape(1, DO), W4, b4.reshape(1, 1))


def _combine(num_flat):
    blk = 2000
    grid = (N // blk,)
    return pl.pallas_call(
        _combine_body,
        grid=grid,
        in_specs=[
            pl.BlockSpec((blk, DC), lambda i: (i, 0)),
            pl.BlockSpec((blk, DC), lambda i: (i + N // blk, 0)),
        ],
        out_specs=pl.BlockSpec((blk, DO), lambda i: (i, 0)),
        out_shape=jax.ShapeDtypeStruct((N, DO), jnp.float32),
    )(num_flat, num_flat)


@functools.cache
def _get_sc_edges():
    return pl.kernel(
        _sc_body,
        out_type=[
            jax.ShapeDtypeStruct((E, DO), jnp.float32),
            jax.ShapeDtypeStruct((NC * N, DC), jnp.float32),
        ],
        mesh=plsc.VectorSubcoreMesh(core_axis_name="c", subcore_axis_name="s"),
        scratch_types=[
            pltpu.VMEM((2, BK), jnp.int32),
            pltpu.VMEM((2, BK), jnp.int32),
            pltpu.VMEM((2, BK), jnp.float32),
            pltpu.VMEM((BK,), jnp.float32),
            pltpu.VMEM((BK,), jnp.float32),
            pltpu.VMEM((BK,), jnp.float32),
            pltpu.VMEM((2, BK, DO), jnp.float32),
            pltpu.VMEM((BK, DC), jnp.float32),
            pltpu.VMEM_SHARED((N, DC), jnp.float32),
            pltpu.SemaphoreType.DMA((2,)),
            pltpu.SemaphoreType.DMA,
            pltpu.SemaphoreType.DMA,
            pltpu.SemaphoreType.DMA,
        ],
        compiler_params=pltpu.CompilerParams(use_tc_tiling_on_sc=False,
                                             needs_layout_passes=False),
    )


def kernel(nodes, edges, senders, receivers, W1, b1, W2, b2, W3, b3, W4, b4,
           W5, b5):
    A, B, C, a, bv = _node_dense(nodes, W1, b1, W2, b2, W4, W5, b5)
    EF0, e0 = _edge_dense(edges, W3, b3, W4, b4)

    Cw = jnp.concatenate(
        [C, jnp.ones((N, 1), jnp.float32), jnp.zeros((N, DC - DO - 1),
                                                     jnp.float32)], axis=1)

    s3 = senders.reshape(NW, NB, BK)
    r3 = receivers.reshape(NW, NB, BK)
    e3 = e0.reshape(NW, NB, BK)

    ef, num_flat = _get_sc_edges()(s3, r3, e3, a.reshape(N), bv.reshape(N),
                                   A, B, Cw, EF0)
    new_nodes = _combine(num_flat)
    return new_nodes, ef


# speedup vs baseline: 1.1949x; 1.0001x over previous
import functools

import jax
import jax.numpy as jnp
from jax import lax
from jax.experimental import pallas as pl
from jax.experimental.pallas import tpu as pltpu
from jax.experimental.pallas import tpu_sc as plsc

N, E, DF, DE, DO = 10000, 320000, 128, 16, 128
DC = DO + 16
NC, NS = 2, 16
NW = NC * NS
EPW = E // NW
BK = 80
NB = EPW // BK
RPT = N // NS


def _node_dense_body(x_ref, w1_ref, b1_ref, w2_ref, b2_ref, w4_ref, w5_ref,
                     b5_ref, A_ref, B_ref, C_ref, a_ref, bv_ref):
    x = x_ref[...]
    A = jnp.dot(x, w1_ref[...], preferred_element_type=jnp.float32) + b1_ref[...]
    B = jnp.dot(x, w2_ref[...], preferred_element_type=jnp.float32) + b2_ref[...]
    C = jnp.dot(x, w5_ref[...], preferred_element_type=jnp.float32) + b5_ref[...]
    A_ref[...] = A
    B_ref[...] = B
    C_ref[...] = C
    w4 = w4_ref[...]
    a_ref[...] = jnp.dot(A, w4, preferred_element_type=jnp.float32)
    bv_ref[...] = jnp.dot(B, w4, preferred_element_type=jnp.float32)


def _edge_dense_body(e_ref, w3_ref, b3_ref, w4_ref, b4_ref, EF0_ref, e0_ref):
    ew = e_ref[...]
    EF0 = jnp.dot(ew, w3_ref[...], preferred_element_type=jnp.float32) + b3_ref[...]
    EF0_ref[...] = EF0
    w34 = jnp.dot(w3_ref[...], w4_ref[...], preferred_element_type=jnp.float32)
    c34 = jnp.dot(b3_ref[...], w4_ref[...], preferred_element_type=jnp.float32)
    e0_ref[...] = (jnp.dot(ew, w34, preferred_element_type=jnp.float32)
                   + c34 + b4_ref[...])


def _combine_body(p0_ref, p1_ref, out_ref):
    acc = p0_ref[...] + p1_ref[...]
    num = acc[:, :DO]
    den = acc[:, DO:DO + 1]
    out_ref[...] = jnp.where(den > 0.0, num / den, 0.0)


def _sc_body(s3, r3, e3, a_hbm, b_hbm, A_hbm, B_hbm, Cw_hbm, EF0_hbm,
             ef_out, num_out,
             s_v, r_v, e_v, av, bv, ex_v, bufE, bufC, acc, sem1, sem2,
             semC, semE):
    cid = lax.axis_index("c")
    sid = lax.axis_index("s")
    wid = cid * NS + sid

    def _zrow(i, _):
        for q in range(DC // 16):
            bufC[i, pl.ds(q * 16, 16)] = jnp.zeros((16,), jnp.float32)
        return 0
    lax.fori_loop(0, BK, _zrow, 0)
    for t in range(RPT // BK):
        pltpu.sync_copy(bufC, acc.at[pl.ds(sid * RPT + t * BK, BK)])
    rem = RPT - (RPT // BK) * BK
    if rem:
        pltpu.sync_copy(bufC.at[pl.ds(0, rem)],
                        acc.at[pl.ds(sid * RPT + (RPT // BK) * BK, rem)])
    plsc.subcore_barrier()

    def issue1(n, slot):
        g = wid * EPW + n * BK
        pltpu.async_copy(s3.at[wid, n], s_v.at[slot], sem1.at[slot])
        pltpu.async_copy(r3.at[wid, n], r_v.at[slot], sem1.at[slot])
        pltpu.async_copy(e3.at[wid, n], e_v.at[slot], sem1.at[slot])
        pltpu.async_copy(EF0_hbm.at[pl.ds(g, BK)], bufE.at[slot],
                         sem1.at[slot])

    def wait1(n, slot):
        g = wid * EPW + n * BK
        pltpu.make_async_copy(s3.at[wid, n], s_v.at[slot],
                              sem1.at[slot]).wait()
        pltpu.make_async_copy(r3.at[wid, n], r_v.at[slot],
                              sem1.at[slot]).wait()
        pltpu.make_async_copy(e3.at[wid, n], e_v.at[slot],
                              sem1.at[slot]).wait()
        pltpu.make_async_copy(EF0_hbm.at[pl.ds(g, BK)], bufE.at[slot],
                              sem1.at[slot]).wait()

    issue1(0, 0)

    def _batch(b, _):
        S = b % 2
        issue1(jnp.minimum(b + 1, NB - 1), (b + 1) % 2)
        wait1(b, S)

        sv = s_v.at[S]
        rv = r_v.at[S]
        be = bufE.at[S]
        pltpu.async_copy(a_hbm.at[sv], av, sem2)
        pltpu.async_copy(b_hbm.at[rv], bv, sem2)
        pltpu.async_copy(Cw_hbm.at[sv], bufC, semC)
        pltpu.async_copy(A_hbm.at[sv], be, semE, add=True)
        pltpu.async_copy(B_hbm.at[rv], be, semE, add=True)
        pltpu.make_async_copy(a_hbm.at[sv], av, sem2).wait()
        pltpu.make_async_copy(b_hbm.at[rv], bv, sem2).wait()

        ev = e_v.at[S]
        for q in range(BK // 16):
            sl = pl.ds(q * 16, 16)
            att = ev[sl] + av[sl] + bv[sl]
            att = jnp.where(att >= 0.0, att, 0.01 * att)
            ex_v[sl] = jnp.exp(att)

        pltpu.make_async_copy(Cw_hbm.at[sv], bufC, semC).wait()

        def _scale(e, _):
            exb = plsc.load_gather(ex_v, [jnp.full((16,), e, jnp.int32)])
            for q in range(DC // 16):
                sl = pl.ds(q * 16, 16)
                bufC[e, sl] = bufC[e, sl] * exb
            return 0
        lax.fori_loop(0, BK, _scale, 0)

        pltpu.make_async_copy(A_hbm.at[sv], be, semE).wait()
        pltpu.make_async_copy(B_hbm.at[rv], be, semE).wait()
        g = wid * EPW + b * BK
        pltpu.async_copy(be, ef_out.at[pl.ds(g, BK)], semE)
        pltpu.sync_copy(bufC, acc.at[rv], add=True)
        pltpu.make_async_copy(be, ef_out.at[pl.ds(g, BK)], semE).wait()
        return 0

    lax.fori_loop(0, NB, _batch, 0)
    wait1(NB - 1, NB % 2)
    plsc.subcore_barrier()

    pltpu.sync_copy(acc.at[pl.ds(sid * RPT, RPT)],
                    num_out.at[pl.ds(cid * N + sid * RPT, RPT)])


def _node_dense(nodes, W1, b1, W2, b2, W4, W5, b5):
    blk = 2000
    grid = (N // blk,)
    full = lambda shape: pl.BlockSpec(shape, lambda i: (0, 0))
    return pl.pallas_call(
        _node_dense_body,
        grid=grid,
        in_specs=[
            pl.BlockSpec((blk, DF), lambda i: (i, 0)),
            full((DF, DO)), full((1, DO)),
            full((DF, DO)), full((1, DO)),
            full((DO, 1)),
            full((DF, DO)), full((1, DO)),
        ],
        out_specs=[
            pl.BlockSpec((blk, DO), lambda i: (i, 0)),
            pl.BlockSpec((blk, DO), lambda i: (i, 0)),
            pl.BlockSpec((blk, DO), lambda i: (i, 0)),
            pl.BlockSpec((blk, 1), lambda i: (i, 0)),
            pl.BlockSpec((blk, 1), lambda i: (i, 0)),
        ],
        out_shape=[
            jax.ShapeDtypeStruct((N, DO), jnp.float32),
            jax.ShapeDtypeStruct((N, DO), jnp.float32),
            jax.ShapeDtypeStruct((N, DO), jnp.float32),
            jax.ShapeDtypeStruct((N, 1), jnp.float32),
            jax.ShapeDtypeStruct((N, 1), jnp.float32),
        ],
    )(nodes, W1, b1.reshape(1, DO), W2, b2.reshape(1, DO), W4, W5,
      b5.reshape(1, DO))


def _edge_dense(edges, W3, b3, W4, b4):
    blk = 3200
    grid = (E // blk,)
    full = lambda shape: pl.BlockSpec(shape, lambda i: (0, 0))
    return pl.pallas_call(
        _edge_dense_body,
        grid=grid,
        in_specs=[
            pl.BlockSpec((blk, DE), lambda i: (i, 0)),
            full((DE, DO)), full((1, DO)),
            full((DO, 1)), full((1, 1)),
        ],
        out_specs=[
            pl.BlockSpec((blk, DO), lambda i: (i, 0)),
            pl.BlockSpec((blk, 1), lambda i: (i, 0)),
        ],
        out_shape=[
            jax.ShapeDtypeStruct((E, DO), jnp.float32),
            jax.ShapeDtypeStruct((E, 1), jnp.float32),
        ],
    )(edges, W3, b3.reshape(1, DO), W4, b4.reshape(1, 1))


def _combine(num_flat):
    blk = 2000
    grid = (N // blk,)
    return pl.pallas_call(
        _combine_body,
        grid=grid,
        in_specs=[
            pl.BlockSpec((blk, DC), lambda i: (i, 0)),
            pl.BlockSpec((blk, DC), lambda i: (i + N // blk, 0)),
        ],
        out_specs=pl.BlockSpec((blk, DO), lambda i: (i, 0)),
        out_shape=jax.ShapeDtypeStruct((N, DO), jnp.float32),
    )(num_flat, num_flat)


@functools.cache
def _get_sc_edges():
    return pl.kernel(
        _sc_body,
        out_type=[
            jax.ShapeDtypeStruct((E, DO), jnp.float32),
            jax.ShapeDtypeStruct((NC * N, DC), jnp.float32),
        ],
        mesh=plsc.VectorSubcoreMesh(core_axis_name="c", subcore_axis_name="s"),
        scratch_types=[
            pltpu.VMEM((2, BK), jnp.int32),
            pltpu.VMEM((2, BK), jnp.int32),
            pltpu.VMEM((2, BK), jnp.float32),
            pltpu.VMEM((BK,), jnp.float32),
            pltpu.VMEM((BK,), jnp.float32),
            pltpu.VMEM((BK,), jnp.float32),
            pltpu.VMEM((2, BK, DO), jnp.float32),
            pltpu.VMEM((BK, DC), jnp.float32),
            pltpu.VMEM_SHARED((N, DC), jnp.float32),
            pltpu.SemaphoreType.DMA((2,)),
            pltpu.SemaphoreType.DMA,
            pltpu.SemaphoreType.DMA,
            pltpu.SemaphoreType.DMA,
        ],
        compiler_params=pltpu.CompilerParams(use_tc_tiling_on_sc=False,
                                             needs_layout_passes=False),
    )


def kernel(nodes, edges, senders, receivers, W1, b1, W2, b2, W3, b3, W4, b4,
           W5, b5):
    A, B, C, a, bv = _node_dense(nodes, W1, b1, W2, b2, W4, W5, b5)
    EF0, e0 = _edge_dense(edges, W3, b3, W4, b4)

    Cw = jnp.concatenate(
        [C, jnp.ones((N, 1), jnp.float32), jnp.zeros((N, DC - DO - 1),
                                                     jnp.float32)], axis=1)

    s3 = senders.reshape(NW, NB, BK)
    r3 = receivers.reshape(NW, NB, BK)
    e3 = e0.reshape(NW, NB, BK)

    ef, num_flat = _get_sc_edges()(s3, r3, e3, a.reshape(N), bv.reshape(N),
                                   A, B, Cw, EF0)
    new_nodes = _combine(num_flat)
    return new_nodes, ef
